# baseline (speedup 1.0000x reference)
import functools

import jax
import jax.numpy as jnp
from jax import lax
from jax.experimental import pallas as pl
from jax.experimental.pallas import tpu as pltpu
from jax.experimental.pallas import tpu_sc as plsc

VOCAB = 1000000
EMBED = 32
BATCH = 4096

_INFO = plsc.get_sparse_core_info()
_NC, _NS = _INFO.num_cores, _INFO.num_subcores
_NW = _NC * _NS
_BPW = BATCH // _NW


def _sc_gather_pair(center_words, context_words, embedding_v, embedding_u):
  mesh = plsc.VectorSubcoreMesh(core_axis_name="c", subcore_axis_name="s")

  @functools.partial(
      pl.kernel,
      mesh=mesh,
      compiler_params=pltpu.CompilerParams(use_tc_tiling_on_sc=False),
      out_type=[
          jax.ShapeDtypeStruct((BATCH, EMBED), jnp.float32),
          jax.ShapeDtypeStruct((BATCH, EMBED), jnp.float32),
      ],
      scratch_types=[
          pltpu.VMEM((_BPW,), jnp.int32),
          pltpu.VMEM((_BPW, EMBED), jnp.float32),
          pltpu.VMEM((_BPW,), jnp.int32),
          pltpu.VMEM((_BPW, EMBED), jnp.float32),
          pltpu.SemaphoreType.DMA,
          pltpu.SemaphoreType.DMA,
      ],
  )
  def gather_kernel(cw_hbm, xw_hbm, v_hbm, u_hbm, outc_hbm, outx_hbm,
                    idx_c, rows_c, idx_x, rows_x, sem_c, sem_x):
    wid = lax.axis_index("s") * _NC + lax.axis_index("c")
    base = wid * _BPW
    pltpu.sync_copy(cw_hbm.at[pl.ds(base, _BPW)], idx_c)
    pltpu.sync_copy(xw_hbm.at[pl.ds(base, _BPW)], idx_x)
    cp_c = pltpu.async_copy(v_hbm.at[idx_c], rows_c, sem_c)
    cp_x = pltpu.async_copy(u_hbm.at[idx_x], rows_x, sem_x)
    cp_c.wait()
    cp_x.wait()
    pltpu.sync_copy(rows_c, outc_hbm.at[pl.ds(base, _BPW)])
    pltpu.sync_copy(rows_x, outx_hbm.at[pl.ds(base, _BPW)])

  return gather_kernel(center_words, context_words, embedding_v, embedding_u)


_RB = 512


def _score_logsoftmax_body(c_ref, ctx_ref, o_ref):
  c = c_ref[...]
  ctx = ctx_ref[...]
  s = lax.dot_general(c, ctx, (((1,), (1,)), ((), ())),
                      preferred_element_type=jnp.float32)
  m = jnp.max(s, axis=1, keepdims=True)
  lse = jnp.log(jnp.sum(jnp.exp(s - m), axis=1, keepdims=True)) + m
  o_ref[...] = s - lse


def _tc_score_logsoftmax(center_embed, context_embed):
  return pl.pallas_call(
      _score_logsoftmax_body,
      grid=(BATCH // _RB,),
      in_specs=[
          pl.BlockSpec((_RB, EMBED), lambda i: (i, 0)),
          pl.BlockSpec((BATCH, EMBED), lambda i: (0, 0)),
      ],
      out_specs=pl.BlockSpec((_RB, BATCH), lambda i: (i, 0)),
      out_shape=jax.ShapeDtypeStruct((BATCH, BATCH), jnp.float32),
  )(center_embed, context_embed)


@jax.jit
def kernel(center_words, context_words, embedding_v, embedding_u):
  center_embed, context_embed = _sc_gather_pair(
      center_words, context_words, embedding_v, embedding_u)
  return _tc_score_logsoftmax(center_embed, context_embed)


# speedup vs baseline: 8.8233x; 8.8233x over previous
import functools

import jax
import jax.numpy as jnp
from jax import lax
from jax.experimental import pallas as pl
from jax.experimental.pallas import tpu as pltpu
from jax.experimental.pallas import tpu_sc as plsc

VOCAB = 1000000
EMBED = 32
BATCH = 4096
LANES = 128

_INFO = plsc.get_sparse_core_info()
_NC, _NS = _INFO.num_cores, _INFO.num_subcores
_NW = _NC * _NS
_BPW = BATCH // _NW
_NBUF = 4


def _sc_gather_pair(center_words, context_words, vp, up):
  mesh = plsc.VectorSubcoreMesh(core_axis_name="c", subcore_axis_name="s")

  @functools.partial(
      pl.kernel,
      mesh=mesh,
      compiler_params=pltpu.CompilerParams(needs_layout_passes=False),
      out_type=[
          jax.ShapeDtypeStruct((EMBED, BATCH), jnp.float32),
          jax.ShapeDtypeStruct((EMBED, BATCH), jnp.float32),
      ],
      scratch_types=[
          pltpu.VMEM((_BPW,), jnp.int32),
          pltpu.VMEM((_BPW,), jnp.int32),
          pltpu.VMEM((_NBUF, EMBED, LANES), jnp.float32),
          pltpu.VMEM((_NBUF, EMBED, LANES), jnp.float32),
          pltpu.VMEM((EMBED, _BPW), jnp.float32),
          pltpu.VMEM((EMBED, _BPW), jnp.float32),
          pltpu.SemaphoreType.DMA,
          pltpu.SemaphoreType.DMA,
          pltpu.SemaphoreType.DMA,
          pltpu.SemaphoreType.DMA,
          pltpu.SemaphoreType.DMA,
          pltpu.SemaphoreType.DMA,
          pltpu.SemaphoreType.DMA,
          pltpu.SemaphoreType.DMA,
      ],
  )
  def gather_kernel(cw_hbm, xw_hbm, vp_hbm, up_hbm, outc_hbm, outx_hbm,
                    idxc_v, idxx_v, winc_v, winx_v,
                    colsc_v, colsx_v,
                    semc0, semc1, semc2, semc3, semx0, semx1, semx2, semx3):
    wid = lax.axis_index("s") * _NC + lax.axis_index("c")
    base = wid * _BPW
    pltpu.sync_copy(cw_hbm.at[pl.ds(base, _BPW)], idxc_v)
    pltpu.sync_copy(xw_hbm.at[pl.ds(base, _BPW)], idxx_v)
    semsc = [semc0, semc1, semc2, semc3]
    semsx = [semx0, semx1, semx2, semx3]
    dvec0 = lax.iota(jnp.int32, 16)
    dvec1 = dvec0 + 16

    def read_idx(idx_v, j):
      chunk = idx_v[pl.ds((j // 16) * 16, 16)]
      return jnp.sum(jnp.where(dvec0 == lax.rem(j, 16), chunk, 0))

    def win_start(i):
      return pl.multiple_of((i // LANES) * LANES, LANES)

    def issue(j, b, idx_v, tab_hbm, win_v, sem):
      i = read_idx(idx_v, j)
      pltpu.async_copy(
          tab_hbm.at[:, pl.ds(win_start(i), LANES)], win_v.at[b], sem)

    def extract(j, b, idx_v, win_v, cols_v):
      o = lax.rem(read_idx(idx_v, j), LANES)
      bvec = jnp.full((16,), b, jnp.int32)
      ovec = jnp.full((16,), o, jnp.int32)
      jvec = jnp.full((16,), j, jnp.int32)
      w0 = plsc.load_gather(win_v, [bvec, dvec0, ovec])
      w1 = plsc.load_gather(win_v, [bvec, dvec1, ovec])
      plsc.store_scatter(cols_v, [dvec0, jvec], w0)
      plsc.store_scatter(cols_v, [dvec1, jvec], w1)

    for b in range(_NBUF):
      issue(b, b, idxc_v, vp_hbm, winc_v, semsc[b])
      issue(b, b, idxx_v, up_hbm, winx_v, semsx[b])

    def group_body(g, carry):
      for b in range(_NBUF):
        j = g * _NBUF + b
        pltpu.make_async_copy(
            vp_hbm.at[:, pl.ds(0, LANES)], winc_v.at[b], semsc[b]).wait()
        extract(j, b, idxc_v, winc_v, colsc_v)
        pltpu.make_async_copy(
            up_hbm.at[:, pl.ds(0, LANES)], winx_v.at[b], semsx[b]).wait()
        extract(j, b, idxx_v, winx_v, colsx_v)

        @pl.when(j + _NBUF < _BPW)
        def _():
          issue(j + _NBUF, b, idxc_v, vp_hbm, winc_v, semsc[b])
          issue(j + _NBUF, b, idxx_v, up_hbm, winx_v, semsx[b])

      return carry

    lax.fori_loop(0, _BPW // _NBUF, group_body, 0)
    pltpu.sync_copy(colsc_v, outc_hbm.at[:, pl.ds(base, _BPW)])
    pltpu.sync_copy(colsx_v, outx_hbm.at[:, pl.ds(base, _BPW)])

  return gather_kernel(center_words, context_words, vp, up)


_RB = 512


def _score_logsoftmax_body(c_ref, ctx_ref, o_ref):
  c = c_ref[...]
  ctx = ctx_ref[...]
  s = lax.dot_general(c, ctx, (((0,), (0,)), ((), ())),
                      preferred_element_type=jnp.float32)
  m = jnp.max(s, axis=1, keepdims=True)
  lse = jnp.log(jnp.sum(jnp.exp(s - m), axis=1, keepdims=True)) + m
  o_ref[...] = s - lse


def _tc_score_logsoftmax(center_embed_t, context_embed_t):
  return pl.pallas_call(
      _score_logsoftmax_body,
      grid=(BATCH // _RB,),
      in_specs=[
          pl.BlockSpec((EMBED, _RB), lambda i: (0, i)),
          pl.BlockSpec((EMBED, BATCH), lambda i: (0, 0)),
      ],
      out_specs=pl.BlockSpec((_RB, BATCH), lambda i: (i, 0)),
      out_shape=jax.ShapeDtypeStruct((BATCH, BATCH), jnp.float32),
  )(center_embed_t, context_embed_t)


@jax.jit
def kernel(center_words, context_words, embedding_v, embedding_u):
  cw = center_words.astype(jnp.int32)
  xw = context_words.astype(jnp.int32)
  cg_t, xg_t = _sc_gather_pair(cw, xw, embedding_v.T, embedding_u.T)
  return _tc_score_logsoftmax(cg_t, xg_t)


# speedup vs baseline: 8.9037x; 1.0091x over previous
import functools

import jax
import jax.numpy as jnp
from jax import lax
from jax.experimental import pallas as pl
from jax.experimental.pallas import tpu as pltpu
from jax.experimental.pallas import tpu_sc as plsc

VOCAB = 1000000
EMBED = 32
BATCH = 4096
LANES = 128

_INFO = plsc.get_sparse_core_info()
_NC, _NS = _INFO.num_cores, _INFO.num_subcores
_NW = _NC * _NS
_BPW = BATCH // _NW
_NBUF = 8


def _sc_gather_pair(center_words, context_words, vp, up):
  mesh = plsc.VectorSubcoreMesh(core_axis_name="c", subcore_axis_name="s")

  @functools.partial(
      pl.kernel,
      mesh=mesh,
      compiler_params=pltpu.CompilerParams(needs_layout_passes=False),
      out_type=[
          jax.ShapeDtypeStruct((EMBED, BATCH), jnp.float32),
          jax.ShapeDtypeStruct((EMBED, BATCH), jnp.float32),
      ],
      scratch_types=[
          pltpu.VMEM((_BPW,), jnp.int32),
          pltpu.VMEM((_BPW,), jnp.int32),
          pltpu.VMEM((_NBUF, EMBED, LANES), jnp.float32),
          pltpu.VMEM((_NBUF, EMBED, LANES), jnp.float32),
          pltpu.VMEM((EMBED, _BPW), jnp.float32),
          pltpu.VMEM((EMBED, _BPW), jnp.float32),
      ] + [pltpu.SemaphoreType.DMA] * (2 * _NBUF),
  )
  def gather_kernel(cw_hbm, xw_hbm, vp_hbm, up_hbm, outc_hbm, outx_hbm,
                    idxc_v, idxx_v, winc_v, winx_v,
                    colsc_v, colsx_v, *sems):
    wid = lax.axis_index("s") * _NC + lax.axis_index("c")
    base = wid * _BPW
    pltpu.sync_copy(cw_hbm.at[pl.ds(base, _BPW)], idxc_v)
    pltpu.sync_copy(xw_hbm.at[pl.ds(base, _BPW)], idxx_v)
    semsc = list(sems[:_NBUF])
    semsx = list(sems[_NBUF:])
    dvec0 = lax.iota(jnp.int32, 16)
    dvec1 = dvec0 + 16

    def read_idx(idx_v, j):
      chunk = idx_v[pl.ds((j // 16) * 16, 16)]
      return jnp.sum(jnp.where(dvec0 == lax.rem(j, 16), chunk, 0))

    def win_start(i):
      return pl.multiple_of((i // LANES) * LANES, LANES)

    def issue(j, b, idx_v, tab_hbm, win_v, sem):
      i = read_idx(idx_v, j)
      pltpu.async_copy(
          tab_hbm.at[:, pl.ds(win_start(i), LANES)], win_v.at[b], sem)

    def extract(j, b, idx_v, win_v, cols_v):
      o = lax.rem(read_idx(idx_v, j), LANES)
      bvec = jnp.full((16,), b, jnp.int32)
      ovec = jnp.full((16,), o, jnp.int32)
      jvec = jnp.full((16,), j, jnp.int32)
      w0 = plsc.load_gather(win_v, [bvec, dvec0, ovec])
      w1 = plsc.load_gather(win_v, [bvec, dvec1, ovec])
      plsc.store_scatter(cols_v, [dvec0, jvec], w0)
      plsc.store_scatter(cols_v, [dvec1, jvec], w1)

    for b in range(_NBUF):
      issue(b, b, idxc_v, vp_hbm, winc_v, semsc[b])
      issue(b, b, idxx_v, up_hbm, winx_v, semsx[b])

    def group_body(g, carry):
      for b in range(_NBUF):
        j = g * _NBUF + b
        pltpu.make_async_copy(
            vp_hbm.at[:, pl.ds(0, LANES)], winc_v.at[b], semsc[b]).wait()
        extract(j, b, idxc_v, winc_v, colsc_v)
        pltpu.make_async_copy(
            up_hbm.at[:, pl.ds(0, LANES)], winx_v.at[b], semsx[b]).wait()
        extract(j, b, idxx_v, winx_v, colsx_v)

        @pl.when(j + _NBUF < _BPW)
        def _():
          issue(j + _NBUF, b, idxc_v, vp_hbm, winc_v, semsc[b])
          issue(j + _NBUF, b, idxx_v, up_hbm, winx_v, semsx[b])

      return carry

    lax.fori_loop(0, _BPW // _NBUF, group_body, 0)
    pltpu.sync_copy(colsc_v, outc_hbm.at[:, pl.ds(base, _BPW)])
    pltpu.sync_copy(colsx_v, outx_hbm.at[:, pl.ds(base, _BPW)])

  return gather_kernel(center_words, context_words, vp, up)


_RB = 512


def _score_logsoftmax_body(c_ref, ctx_ref, o_ref):
  c = c_ref[...]
  ctx = ctx_ref[...]
  s = lax.dot_general(c, ctx, (((0,), (0,)), ((), ())),
                      preferred_element_type=jnp.float32)
  m = jnp.max(s, axis=1, keepdims=True)
  lse = jnp.log(jnp.sum(jnp.exp(s - m), axis=1, keepdims=True)) + m
  o_ref[...] = s - lse


def _tc_score_logsoftmax(center_embed_t, context_embed_t):
  return pl.pallas_call(
      _score_logsoftmax_body,
      grid=(BATCH // _RB,),
      in_specs=[
          pl.BlockSpec((EMBED, _RB), lambda i: (0, i)),
          pl.BlockSpec((EMBED, BATCH), lambda i: (0, 0)),
      ],
      out_specs=pl.BlockSpec((_RB, BATCH), lambda i: (i, 0)),
      out_shape=jax.ShapeDtypeStruct((BATCH, BATCH), jnp.float32),
  )(center_embed_t, context_embed_t)


@jax.jit
def kernel(center_words, context_words, embedding_v, embedding_u):
  cw = center_words.astype(jnp.int32)
  xw = context_words.astype(jnp.int32)
  cg_t, xg_t = _sc_gather_pair(cw, xw, embedding_v.T, embedding_u.T)
  return _tc_score_logsoftmax(cg_t, xg_t)
